# Initial kernel scaffold; baseline (speedup 1.0000x reference)
import functools

import jax
import jax.numpy as jnp
from jax import lax
from jax.experimental import pallas as pl
from jax.experimental.pallas import tpu as pltpu
from jax.experimental.pallas import tpu_sc as plsc

_INFO = plsc.get_sparse_core_info()
_NC = _INFO.num_cores
_NS = _INFO.num_subcores
_NW = _NC * _NS

_CHUNK = 128
_GPC = 5
_GROUP = _CHUNK * _GPC


@functools.partial(jax.jit, static_argnums=(2, 3))
def _sc_gather(idx, table, n_groups, d):
    mesh = plsc.VectorSubcoreMesh(core_axis_name="c", subcore_axis_name="s")
    n = idx.shape[0] * idx.shape[1] * idx.shape[2]
    bpw = n // _NW

    @functools.partial(
        pl.kernel,
        out_type=jax.ShapeDtypeStruct((n, d), table.dtype),
        mesh=mesh,
        scratch_types=[
            pltpu.VMEM((bpw // _CHUNK, _CHUNK), jnp.int32),
            pltpu.VMEM((_GROUP, d), table.dtype),
            pltpu.SemaphoreType.DMA,
        ],
    )
    def body(idx_hbm, tbl_hbm, out_hbm, idx_v, rows_v, gsem):
        wid = lax.axis_index("s") * _NC + lax.axis_index("c")
        pltpu.sync_copy(idx_hbm.at[wid], idx_v)
        base = wid * bpw

        def group(g, carry):
            descs = [
                pltpu.async_copy(
                    tbl_hbm.at[idx_v.at[g * _GPC + j]],
                    rows_v.at[pl.ds(j * _CHUNK, _CHUNK)],
                    gsem,
                )
                for j in range(_GPC)
            ]
            for dsc in descs:
                dsc.wait()
            pltpu.sync_copy(
                rows_v, out_hbm.at[pl.ds(base + g * _GROUP, _GROUP)]
            )
            return carry

        lax.fori_loop(0, n_groups, group, 0)

    return body(idx, table)


def kernel(indices, table):
    b, l = indices.shape
    v, d = table.shape
    n = b * l
    idx = indices.astype(jnp.int32).reshape(_NW, (n // _NW) // _CHUNK, _CHUNK)
    out = _sc_gather(idx, table, (n // _NW) // _GROUP, d)
    return out.reshape(b, l, d)


# speedup vs baseline: 4.5686x; 4.5686x over previous
import functools

import jax
import jax.numpy as jnp
from jax import lax
from jax.experimental import pallas as pl
from jax.experimental.pallas import tpu as pltpu
from jax.experimental.pallas import tpu_sc as plsc

_INFO = plsc.get_sparse_core_info()
_NC = _INFO.num_cores
_NS = _INFO.num_subcores
_NW = _NC * _NS

_CHUNK = 128
_GPC = 5
_GROUP = _CHUNK * _GPC


@functools.partial(jax.jit, static_argnums=(2, 3))
def _sc_gather(idx, table, n_groups, d):
    mesh = plsc.VectorSubcoreMesh(core_axis_name="c", subcore_axis_name="s")
    n = idx.shape[0] * idx.shape[1] * idx.shape[2]
    bpw = n // _NW

    @functools.partial(
        pl.kernel,
        out_type=jax.ShapeDtypeStruct((n, d), table.dtype),
        mesh=mesh,
        scratch_types=[
            pltpu.VMEM((bpw // _CHUNK, _CHUNK), jnp.int32),
            pltpu.VMEM((_GROUP, d), table.dtype),
            pltpu.SemaphoreType.DMA,
        ],
        compiler_params=pltpu.CompilerParams(use_tc_tiling_on_sc=False),
    )
    def body(idx_hbm, tbl_hbm, out_hbm, idx_v, rows_v, gsem):
        wid = lax.axis_index("s") * _NC + lax.axis_index("c")
        pltpu.sync_copy(idx_hbm.at[wid], idx_v)
        base = wid * bpw

        def group(g, carry):
            descs = [
                pltpu.async_copy(
                    tbl_hbm.at[idx_v.at[g * _GPC + j]],
                    rows_v.at[pl.ds(j * _CHUNK, _CHUNK)],
                    gsem,
                )
                for j in range(_GPC)
            ]
            for dsc in descs:
                dsc.wait()
            pltpu.sync_copy(
                rows_v, out_hbm.at[pl.ds(base + g * _GROUP, _GROUP)]
            )
            return carry

        lax.fori_loop(0, n_groups, group, 0)

    return body(idx, table)


def kernel(indices, table):
    b, l = indices.shape
    v, d = table.shape
    n = b * l
    idx = indices.astype(jnp.int32).reshape(_NW, (n // _NW) // _CHUNK, _CHUNK)
    out = _sc_gather(idx, table, (n // _NW) // _GROUP, d)
    return out.reshape(b, l, d)


# speedup vs baseline: 4.6329x; 1.0141x over previous
import functools

import jax
import jax.numpy as jnp
from jax import lax
from jax.experimental import pallas as pl
from jax.experimental.pallas import tpu as pltpu
from jax.experimental.pallas import tpu_sc as plsc

_INFO = plsc.get_sparse_core_info()
_NC = _INFO.num_cores
_NS = _INFO.num_subcores
_NW = _NC * _NS

_CHUNK = 128
_GPC = 5
_GROUP = _CHUNK * _GPC


@functools.partial(jax.jit, static_argnums=(2, 3))
def _sc_gather(idx, table, n_groups, d):
    mesh = plsc.VectorSubcoreMesh(core_axis_name="c", subcore_axis_name="s")
    n = idx.shape[0] * idx.shape[1] * idx.shape[2]
    bpw = n // _NW

    @functools.partial(
        pl.kernel,
        out_type=jax.ShapeDtypeStruct((n, d), table.dtype),
        mesh=mesh,
        scratch_types=[
            pltpu.VMEM((bpw // _CHUNK, _CHUNK), jnp.int32),
            pltpu.VMEM((2, _GROUP, d), table.dtype),
            pltpu.SemaphoreType.DMA,
            pltpu.SemaphoreType.DMA,
            pltpu.SemaphoreType.DMA,
        ],
        compiler_params=pltpu.CompilerParams(use_tc_tiling_on_sc=False),
    )
    def body(idx_hbm, tbl_hbm, out_hbm, idx_v, rows_v, gsem, wsem0, wsem1):
        wid = lax.axis_index("s") * _NC + lax.axis_index("c")
        pltpu.sync_copy(idx_hbm.at[wid], idx_v)
        base = wid * bpw
        wsems = (wsem0, wsem1)

        def gather_group(g, b):
            descs = [
                pltpu.async_copy(
                    tbl_hbm.at[idx_v.at[g * _GPC + j]],
                    rows_v.at[b].at[pl.ds(j * _CHUNK, _CHUNK)],
                    gsem,
                )
                for j in range(_GPC)
            ]
            for dsc in descs:
                dsc.wait()

        def start_writeback(g, b):
            pltpu.async_copy(
                rows_v.at[b],
                out_hbm.at[pl.ds(base + g * _GROUP, _GROUP)],
                wsems[b],
            )

        def drain_writeback(g, b):
            pltpu.make_async_copy(
                rows_v.at[b],
                out_hbm.at[pl.ds(base + g * _GROUP, _GROUP)],
                wsems[b],
            ).wait()

        for b in range(2):
            gather_group(b, b)
            start_writeback(b, b)

        def outer(o, carry):
            for b in range(2):
                g = o * 2 + b
                drain_writeback(g - 2, b)
                gather_group(g, b)
                start_writeback(g, b)
            return carry

        lax.fori_loop(1, n_groups // 2, outer, 0)

        for b in range(2):
            drain_writeback(n_groups - 2 + b, b)

    return body(idx, table)


def kernel(indices, table):
    b, l = indices.shape
    v, d = table.shape
    n = b * l
    idx = indices.astype(jnp.int32).reshape(_NW, (n // _NW) // _CHUNK, _CHUNK)
    out = _sc_gather(idx, table, (n // _NW) // _GROUP, d)
    return out.reshape(b, l, d)


# speedup vs baseline: 6.2594x; 1.3511x over previous
import functools

import jax
import jax.numpy as jnp
from jax import lax
from jax.experimental import pallas as pl
from jax.experimental.pallas import tpu as pltpu
from jax.experimental.pallas import tpu_sc as plsc

_INFO = plsc.get_sparse_core_info()
_NC = _INFO.num_cores
_NS = _INFO.num_subcores
_NW = _NC * _NS


@jax.jit
def _sc_lookup(idx_t, tbl_t):
    l_len, b_len = idx_t.shape
    d_len, v_len = tbl_t.shape
    rounds = d_len // _NW
    mesh = plsc.VectorSubcoreMesh(core_axis_name="c", subcore_axis_name="s")

    @functools.partial(
        pl.kernel,
        out_type=jax.ShapeDtypeStruct((l_len, d_len, b_len), tbl_t.dtype),
        mesh=mesh,
        scratch_types=[
            pltpu.VMEM((1, v_len), tbl_t.dtype),
            pltpu.VMEM((1, b_len), jnp.int32),
            pltpu.VMEM((1, b_len), tbl_t.dtype),
        ],
        compiler_params=pltpu.CompilerParams(
            use_tc_tiling_on_sc=True, needs_layout_passes=False
        ),
    )
    def body(idx_hbm, tbl_hbm, out_hbm, row_v, idx_v, res_v):
        wid = lax.axis_index("s") * _NC + lax.axis_index("c")
        zeros = jnp.zeros((16,), jnp.int32)

        for r in range(rounds):
            dd = r * _NW + wid
            pltpu.sync_copy(tbl_hbm.at[pl.ds(dd, 1)], row_v)

            def per_l(l, carry):
                pltpu.sync_copy(idx_hbm.at[pl.ds(l, 1)], idx_v)

                @plsc.parallel_loop(0, b_len, 16, unroll=8)
                def gather16(k):
                    iv = idx_v[0, pl.ds(k, 16)]
                    res_v[0, pl.ds(k, 16)] = plsc.load_gather(
                        row_v, [zeros, iv]
                    )
                pltpu.sync_copy(res_v, out_hbm.at[l, pl.ds(dd, 1)])
                return carry

            lax.fori_loop(0, l_len, per_l, 0)

    return body(idx_t, tbl_t)


def kernel(indices, table):
    idx_t = jnp.transpose(indices.astype(jnp.int32))
    tbl_t = jnp.transpose(table)
    out_t = _sc_lookup(idx_t, tbl_t)
    return jnp.transpose(out_t, (2, 0, 1))


# speedup vs baseline: 8.4751x; 1.3540x over previous
import functools

import jax
import jax.numpy as jnp
from jax import lax
from jax.experimental import pallas as pl
from jax.experimental.pallas import tpu as pltpu
from jax.experimental.pallas import tpu_sc as plsc

_INFO = plsc.get_sparse_core_info()
_NC = _INFO.num_cores
_NS = _INFO.num_subcores
_NW = _NC * _NS


@jax.jit
def _sc_lookup(idx_t, tbl_t):
    l_len, b_len = idx_t.shape
    d_len, v_len = tbl_t.shape
    rounds = d_len // _NW
    mesh = plsc.VectorSubcoreMesh(core_axis_name="c", subcore_axis_name="s")

    @functools.partial(
        pl.kernel,
        out_type=jax.ShapeDtypeStruct((l_len, d_len, b_len), tbl_t.dtype),
        mesh=mesh,
        scratch_types=[
            pltpu.VMEM((1, v_len), tbl_t.dtype),
            pltpu.VMEM((1, b_len), jnp.int32),
            pltpu.VMEM((1, b_len), jnp.int32),
            pltpu.VMEM((1, b_len), tbl_t.dtype),
            pltpu.VMEM((1, b_len), tbl_t.dtype),
            pltpu.SemaphoreType.DMA,
            pltpu.SemaphoreType.DMA,
            pltpu.SemaphoreType.DMA,
        ],
        compiler_params=pltpu.CompilerParams(
            use_tc_tiling_on_sc=True, needs_layout_passes=False
        ),
    )
    def body(idx_hbm, tbl_hbm, out_hbm, row_v, idx_v0, idx_v1,
             res_v0, res_v1, isem, wsem0, wsem1):
        wid = lax.axis_index("s") * _NC + lax.axis_index("c")
        zeros = jnp.zeros((16,), jnp.int32)
        idx_bufs = (idx_v0, idx_v1)
        res_bufs = (res_v0, res_v1)
        wsems = (wsem0, wsem1)

        for r in range(rounds):
            dd = r * _NW + wid
            pltpu.sync_copy(tbl_hbm.at[pl.ds(dd, 1)], row_v)
            pltpu.async_copy(idx_hbm.at[pl.ds(0, 1)], idx_bufs[0], isem)

            def pair(o, carry):
                for par in range(2):
                    l = o * 2 + par
                    pltpu.make_async_copy(
                        idx_hbm.at[pl.ds(l, 1)], idx_bufs[par], isem
                    ).wait()

                    @pl.when(l + 1 < l_len)
                    def _prefetch():
                        pltpu.async_copy(
                            idx_hbm.at[pl.ds(l + 1, 1)],
                            idx_bufs[1 - par], isem,
                        )

                    @pl.when(l >= 2)
                    def _drain():
                        pltpu.make_async_copy(
                            res_bufs[par],
                            out_hbm.at[l - 2, pl.ds(dd, 1)],
                            wsems[par],
                        ).wait()

                    ib = idx_bufs[par]
                    rb = res_bufs[par]

                    @plsc.parallel_loop(0, b_len, 16, unroll=8)
                    def gather16(k):
                        iv = ib[0, pl.ds(k, 16)]
                        rb[0, pl.ds(k, 16)] = plsc.load_gather(
                            row_v, [zeros, iv]
                        )

                    pltpu.async_copy(
                        rb, out_hbm.at[l, pl.ds(dd, 1)], wsems[par]
                    )
                return carry

            lax.fori_loop(0, l_len // 2, pair, 0)

            for par in range(2):
                pltpu.make_async_copy(
                    res_bufs[par],
                    out_hbm.at[l_len - 2 + par, pl.ds(dd, 1)],
                    wsems[par],
                ).wait()

    return body(idx_t, tbl_t)


def kernel(indices, table):
    idx_t = jnp.transpose(indices.astype(jnp.int32))
    tbl_t = jnp.transpose(table)
    out_t = _sc_lookup(idx_t, tbl_t)
    return jnp.transpose(out_t, (2, 0, 1))


# speedup vs baseline: 11.1350x; 1.3138x over previous
import functools

import jax
import jax.numpy as jnp
from jax import lax
from jax.experimental import pallas as pl
from jax.experimental.pallas import tpu as pltpu
from jax.experimental.pallas import tpu_sc as plsc

_INFO = plsc.get_sparse_core_info()
_NC = _INFO.num_cores
_NS = _INFO.num_subcores
_NW = _NC * _NS


@jax.jit
def _sc_lookup(idx_t, tbl_t):
    l_len, b_len = idx_t.shape
    d_len, v_len = tbl_t.shape
    rounds = d_len // _NW
    mesh = plsc.VectorSubcoreMesh(core_axis_name="c", subcore_axis_name="s")

    @functools.partial(
        pl.kernel,
        out_type=jax.ShapeDtypeStruct((l_len, d_len, b_len), tbl_t.dtype),
        mesh=mesh,
        scratch_types=[
            pltpu.VMEM((1, v_len), tbl_t.dtype),
            pltpu.VMEM((1, b_len), tbl_t.dtype),
            pltpu.VMEM((1, b_len), tbl_t.dtype),
            pltpu.VMEM((1, b_len), tbl_t.dtype),
            pltpu.VMEM((1, b_len), tbl_t.dtype),
            pltpu.VMEM((1, b_len), tbl_t.dtype),
            pltpu.SemaphoreType.DMA,
            pltpu.SemaphoreType.DMA,
            pltpu.SemaphoreType.DMA,
            pltpu.SemaphoreType.DMA,
            pltpu.SemaphoreType.DMA,
            pltpu.SemaphoreType.DMA,
        ],
        compiler_params=pltpu.CompilerParams(
            use_tc_tiling_on_sc=True, needs_layout_passes=False
        ),
    )
    def body(idx_hbm, tbl_hbm, out_hbm, row_v, b0, b1, b2, b3, b4,
             isem, w0, w1, w2, w3, w4):
        wid = lax.axis_index("s") * _NC + lax.axis_index("c")
        zeros = jnp.zeros((16,), jnp.int32)
        bufs = (b0, b1, b2, b3, b4)
        wsems = (w0, w1, w2, w3, w4)
        depth = len(bufs)

        for r in range(rounds):
            dd = r * _NW + wid
            pltpu.sync_copy(tbl_hbm.at[pl.ds(dd, 1)], row_v)
            for m in range(3):
                pltpu.async_copy(idx_hbm.at[pl.ds(m, 1)], bufs[m], isem)

            def block(o, carry):
                for m in range(depth):
                    l = o * depth + m
                    pltpu.make_async_copy(
                        idx_hbm.at[pl.ds(l, 1)], bufs[m], isem
                    ).wait()
                    buf = bufs[m]

                    @plsc.parallel_loop(0, b_len, 16, unroll=8)
                    def gather16(k):
                        iv = plsc.bitcast(buf[0, pl.ds(k, 16)], jnp.int32)
                        buf[0, pl.ds(k, 16)] = plsc.load_gather(
                            row_v, [zeros, iv]
                        )

                    pltpu.async_copy(
                        buf, out_hbm.at[l, pl.ds(dd, 1)], wsems[m]
                    )

                    m2 = (m + 3) % depth

                    @pl.when(l >= 2)
                    def _drain():
                        pltpu.make_async_copy(
                            bufs[m2], out_hbm.at[l - 2, pl.ds(dd, 1)],
                            wsems[m2],
                        ).wait()

                    @pl.when(l + 3 < l_len)
                    def _prefetch():
                        pltpu.async_copy(
                            idx_hbm.at[pl.ds(l + 3, 1)], bufs[m2], isem
                        )
                return carry

            lax.fori_loop(0, l_len // depth, block, 0)

            for l in (l_len - 2, l_len - 1):
                pltpu.make_async_copy(
                    bufs[l % depth], out_hbm.at[l, pl.ds(dd, 1)],
                    wsems[l % depth],
                ).wait()

    return body(idx_t, tbl_t)


def kernel(indices, table):
    idx_t = lax.bitcast_convert_type(
        jnp.transpose(indices.astype(jnp.int32)), jnp.float32
    )
    tbl_t = jnp.transpose(table)
    out_t = _sc_lookup(idx_t, tbl_t)
    return jnp.transpose(out_t, (2, 0, 1))
